# Initial kernel scaffold; baseline (speedup 1.0000x reference)
import jax
import jax.numpy as jnp
from jax.experimental import pallas as pl


def kernel(feature, edge_index, edge_weight, W, b):
    raise NotImplementedError("write your pallas kernel here")


# speedup vs baseline: 3.0528x; 3.0528x over previous
import functools

import jax
import jax.numpy as jnp
from jax import lax
from jax.experimental import pallas as pl
from jax.experimental.pallas import tpu as pltpu
from jax.experimental.pallas import tpu_sc as plsc

NC = 2
NS = 16
NW = NC * NS
LANES = 16
ROWLEN = 144


def _sc_partials(n, e, d, feature, src, dst, wgt, zeros):
    epw = e // NW
    chunk = 80
    nchunk = epw // chunk
    rows_pt = n // NS

    mesh = plsc.VectorSubcoreMesh(core_axis_name="c", subcore_axis_name="s")

    @functools.partial(
        pl.kernel,
        out_type=jax.ShapeDtypeStruct((NC, n, ROWLEN), jnp.float32),
        mesh=mesh,
        compiler_params=pltpu.CompilerParams(
            use_tc_tiling_on_sc=False, needs_layout_passes=False),
        scratch_types=[
            pltpu.VMEM((chunk,), jnp.int32),
            pltpu.VMEM((chunk,), jnp.int32),
            pltpu.VMEM((chunk,), jnp.float32),
            pltpu.VMEM((chunk, d), jnp.float32),
            pltpu.VMEM((chunk, ROWLEN), jnp.float32),
            pltpu.VMEM_SHARED((n, ROWLEN), jnp.float32),
            pltpu.SemaphoreType.DMA,
        ],
    )
    def sc_kernel(feat_hbm, src_hbm, dst_hbm, w_hbm, zero_hbm, out_hbm,
                  src_v, dst_v, w_v, rows_v, wrows_v, acc_sh, sem):
        cid = lax.axis_index("c")
        sid = lax.axis_index("s")
        wid = sid * NC + cid

        ones16 = jnp.where(lax.iota(jnp.int32, LANES) == 0,
                           jnp.float32(1.0), jnp.float32(0.0))

        def init_deg(k, carry):
            wrows_v[k, pl.ds(d, LANES)] = ones16
            return carry
        lax.fori_loop(0, chunk, init_deg, None)

        r0 = sid * rows_pt
        pltpu.sync_copy(zero_hbm.at[pl.ds(r0, rows_pt), :],
                        acc_sh.at[pl.ds(r0, rows_pt), :])
        plsc.subcore_barrier()

        def chunk_body(ci, carry):
            base = pl.multiple_of(wid * epw + ci * chunk, 8)
            pltpu.sync_copy(src_hbm.at[pl.ds(base, chunk)], src_v)
            pltpu.sync_copy(dst_hbm.at[pl.ds(base, chunk)], dst_v)
            pltpu.sync_copy(w_hbm.at[pl.ds(base, chunk)], w_v)
            pltpu.async_copy(feat_hbm.at[src_v], rows_v, sem).wait()

            def edge_body(k, c2):
                idx = jnp.full((LANES,), k, jnp.int32)
                wb = plsc.load_gather(w_v, [idx])
                for j in range(d // LANES):
                    wrows_v[k, pl.ds(j * LANES, LANES)] = (
                        rows_v[k, pl.ds(j * LANES, LANES)] * wb)
                return c2
            lax.fori_loop(0, chunk, edge_body, None)

            pltpu.sync_copy(wrows_v, acc_sh.at[dst_v], add=True)
            return carry
        lax.fori_loop(0, nchunk, chunk_body, None)

        plsc.subcore_barrier()
        pltpu.sync_copy(acc_sh.at[pl.ds(r0, rows_pt), :],
                        out_hbm.at[cid, pl.ds(r0, rows_pt), :])

    return sc_kernel(feature, src, dst, wgt, zeros)


def _tc_combine(n, d, partial, feature, w1t, w2t, b2d):
    blk = 1000

    def body(part_ref, f_ref, w1_ref, w2_ref, b_ref, o_ref):
        p = part_ref[0] + part_ref[1]
        s = p[:, :d]
        deg = p[:, d:d + 1]
        r = s / jnp.maximum(deg, 1.0)
        acc = jnp.dot(f_ref[...], w1_ref[...],
                      preferred_element_type=jnp.float32)
        acc = acc + jnp.dot(r, w2_ref[...],
                            preferred_element_type=jnp.float32)
        o_ref[...] = jnp.maximum(acc + b_ref[...], 0.0)

    return pl.pallas_call(
        body,
        grid=(n // blk,),
        in_specs=[
            pl.BlockSpec((NC, blk, ROWLEN), lambda i: (0, i, 0)),
            pl.BlockSpec((blk, d), lambda i: (i, 0)),
            pl.BlockSpec((d, d), lambda i: (0, 0)),
            pl.BlockSpec((d, d), lambda i: (0, 0)),
            pl.BlockSpec((1, d), lambda i: (0, 0)),
        ],
        out_specs=pl.BlockSpec((blk, d), lambda i: (i, 0)),
        out_shape=jax.ShapeDtypeStruct((n, d), jnp.float32),
    )(partial, feature, w1t, w2t, b2d)


@jax.jit
def kernel(feature, edge_index, edge_weight, W, b):
    n, d = feature.shape
    e = edge_index.shape[1]
    src = edge_index[0]
    dst = edge_index[1]

    c = feature[src[0]]
    w1t = W[:, :d].T
    w2t = (W[:, d:] * c[None, :]).T
    b2d = b.reshape(1, d)
    zeros = jnp.zeros((n, ROWLEN), jnp.float32)

    partial = _sc_partials(n, e, d, feature, src, dst, edge_weight, zeros)
    return _tc_combine(n, d, partial, feature, w1t, w2t, b2d)


# speedup vs baseline: 11.0376x; 3.6156x over previous
import functools

import jax
import jax.numpy as jnp
from jax import lax
from jax.experimental import pallas as pl
from jax.experimental.pallas import tpu as pltpu
from jax.experimental.pallas import tpu_sc as plsc

NC = 2
NS = 16
NW = NC * NS
LANES = 16
ROWLEN = 144


def _sc_partials(n, e, d, feature, srcr, dstr, wr, zeros_s, zeros_d):
    epw = e // NW
    chunk = 80
    nchunk = epw // chunk
    stage = 25
    nstage = nchunk // stage
    npair = stage // 2
    rows_pt = n // NS
    jblocks = d // LANES

    mesh = plsc.VectorSubcoreMesh(core_axis_name="c", subcore_axis_name="s")

    @functools.partial(
        pl.kernel,
        out_type=(jax.ShapeDtypeStruct((NC, n, d), jnp.float32),
                  jax.ShapeDtypeStruct((NC, n, LANES), jnp.float32)),
        mesh=mesh,
        compiler_params=pltpu.CompilerParams(
            use_tc_tiling_on_sc=False, needs_layout_passes=False),
        scratch_types=[
            pltpu.VMEM((stage, chunk), jnp.int32),
            pltpu.VMEM((stage, chunk), jnp.int32),
            pltpu.VMEM((stage, chunk), jnp.float32),
            pltpu.VMEM((2, chunk, d), jnp.float32),
            pltpu.VMEM((chunk, LANES), jnp.float32),
            pltpu.VMEM_SHARED((n, d), jnp.float32),
            pltpu.VMEM_SHARED((n, LANES), jnp.float32),
            pltpu.SemaphoreType.DMA,
            pltpu.SemaphoreType.DMA,
            pltpu.SemaphoreType.DMA,
        ],
    )
    def sc_kernel(feat_hbm, src_hbm, dst_hbm, w_hbm, zs_hbm, zd_hbm,
                  out_s_hbm, out_d_hbm,
                  src_v, dst_v, w_v, rows_v, ones_v, acc_sh, deg_sh,
                  sem0, sem1, semdeg):
        cid = lax.axis_index("c")
        sid = lax.axis_index("s")
        wid = sid * NC + cid
        sems = (sem0, sem1)

        ones16 = jnp.where(lax.iota(jnp.int32, LANES) == 0,
                           jnp.float32(1.0), jnp.float32(0.0))

        def init_ones(k, carry):
            ones_v[k, pl.ds(0, LANES)] = ones16
            return carry
        lax.fori_loop(0, chunk, init_ones, None)

        r0 = sid * rows_pt
        pltpu.sync_copy(zs_hbm.at[pl.ds(r0, rows_pt), :],
                        acc_sh.at[pl.ds(r0, rows_pt), :])
        pltpu.sync_copy(zd_hbm.at[pl.ds(r0, rows_pt), :],
                        deg_sh.at[pl.ds(r0, rows_pt), :])
        plsc.subcore_barrier()

        def issue_gather(ci, b):
            pltpu.async_copy(feat_hbm.at[src_v.at[ci]], rows_v.at[b], sems[b])

        def wait_gather(ci, b):
            pltpu.make_async_copy(feat_hbm.at[src_v.at[ci]],
                                  rows_v.at[b], sems[b]).wait()

        def compute_scatter(ci, b):
            def group_body(g, carry):
                k0 = g * LANES
                wg = w_v[ci, pl.ds(k0, LANES)]
                for j in range(LANES):
                    wb = lax.gather(
                        wg, jnp.full((LANES, 1), j, jnp.int32),
                        lax.GatherDimensionNumbers(
                            offset_dims=(), collapsed_slice_dims=(0,),
                            start_index_map=(0,)),
                        (1,), mode=lax.GatherScatterMode.PROMISE_IN_BOUNDS)
                    k = k0 + j
                    for jj in range(jblocks):
                        rows_v[b, k, pl.ds(jj * LANES, LANES)] = (
                            rows_v[b, k, pl.ds(jj * LANES, LANES)] * wb)
                return carry
            lax.fori_loop(0, chunk // LANES, group_body, None)
            pltpu.async_copy(ones_v, deg_sh.at[dst_v.at[ci]], semdeg, add=True)
            pltpu.sync_copy(rows_v.at[b], acc_sh.at[dst_v.at[ci]], add=True)

        def stage_body(s, carry):
            sb = s * stage
            pltpu.sync_copy(src_hbm.at[wid, pl.ds(sb, stage), :], src_v)
            pltpu.sync_copy(dst_hbm.at[wid, pl.ds(sb, stage), :], dst_v)
            pltpu.sync_copy(w_hbm.at[wid, pl.ds(sb, stage), :], w_v)

            issue_gather(0, 0)

            def pair_body(p, c2):
                ci0 = 2 * p
                wait_gather(ci0, 0)
                issue_gather(ci0 + 1, 1)
                compute_scatter(ci0, 0)
                wait_gather(ci0 + 1, 1)
                issue_gather(ci0 + 2, 0)
                compute_scatter(ci0 + 1, 1)
                return c2
            lax.fori_loop(0, npair, pair_body, None)

            wait_gather(stage - 1, 0)
            compute_scatter(stage - 1, 0)

            def deg_drain(ci, c2):
                pltpu.make_async_copy(ones_v, deg_sh.at[dst_v.at[0]],
                                      semdeg).wait()
                return c2
            lax.fori_loop(0, stage, deg_drain, None)
            return carry
        lax.fori_loop(0, nstage, stage_body, None)

        plsc.subcore_barrier()
        pltpu.sync_copy(acc_sh.at[pl.ds(r0, rows_pt), :],
                        out_s_hbm.at[cid, pl.ds(r0, rows_pt), :])
        pltpu.sync_copy(deg_sh.at[pl.ds(r0, rows_pt), :],
                        out_d_hbm.at[cid, pl.ds(r0, rows_pt), :])

    return sc_kernel(feature, srcr, dstr, wr, zeros_s, zeros_d)


def _tc_combine(n, d, psum, pdeg, feature, w1t, w2t, b2d):
    blk = 1000

    def body(ps_ref, pd_ref, f_ref, w1_ref, w2_ref, b_ref, o_ref):
        s = ps_ref[0] + ps_ref[1]
        deg = pd_ref[0, :, 0:1] + pd_ref[1, :, 0:1]
        r = s / jnp.maximum(deg, 1.0)
        acc = jnp.dot(f_ref[...], w1_ref[...],
                      preferred_element_type=jnp.float32)
        acc = acc + jnp.dot(r, w2_ref[...],
                            preferred_element_type=jnp.float32)
        o_ref[...] = jnp.maximum(acc + b_ref[...], 0.0)

    return pl.pallas_call(
        body,
        grid=(n // blk,),
        in_specs=[
            pl.BlockSpec((NC, blk, d), lambda i: (0, i, 0)),
            pl.BlockSpec((NC, blk, LANES), lambda i: (0, i, 0)),
            pl.BlockSpec((blk, d), lambda i: (i, 0)),
            pl.BlockSpec((d, d), lambda i: (0, 0)),
            pl.BlockSpec((d, d), lambda i: (0, 0)),
            pl.BlockSpec((1, d), lambda i: (0, 0)),
        ],
        out_specs=pl.BlockSpec((blk, d), lambda i: (i, 0)),
        out_shape=jax.ShapeDtypeStruct((n, d), jnp.float32),
    )(psum, pdeg, feature, w1t, w2t, b2d)


@jax.jit
def kernel(feature, edge_index, edge_weight, W, b):
    n, d = feature.shape
    e = edge_index.shape[1]
    src = edge_index[0]
    dst = edge_index[1]

    c = feature[src[0]]
    w1t = W[:, :d].T
    w2t = (W[:, d:] * c[None, :]).T
    b2d = b.reshape(1, d)
    zeros_s = jnp.zeros((n, d), jnp.float32)
    zeros_d = jnp.zeros((n, LANES), jnp.float32)

    epw = e // NW
    chunk = 80
    srcr = src.reshape(NW, epw // chunk, chunk)
    dstr = dst.reshape(NW, epw // chunk, chunk)
    wr = edge_weight.reshape(NW, epw // chunk, chunk)

    psum, pdeg = _sc_partials(n, e, d, feature, srcr, dstr, wr,
                              zeros_s, zeros_d)
    return _tc_combine(n, d, psum, pdeg, feature, w1t, w2t, b2d)


# speedup vs baseline: 12.7142x; 1.1519x over previous
import functools

import jax
import jax.numpy as jnp
from jax import lax
from jax.experimental import pallas as pl
from jax.experimental.pallas import tpu as pltpu
from jax.experimental.pallas import tpu_sc as plsc

NC = 2
NS = 16
NW = NC * NS
LANES = 16
ROWLEN = 144


def _sc_partials(n, e, d, feature, srcr, dstr, wr, zeros_s, zeros_d):
    epw = e // NW
    chunk = 80
    nchunk = epw // chunk
    stage = 25
    nstage = nchunk // stage
    npair = stage // 2
    rows_pt = n // NS
    jblocks = d // LANES

    mesh = plsc.VectorSubcoreMesh(core_axis_name="c", subcore_axis_name="s")

    @functools.partial(
        pl.kernel,
        out_type=(jax.ShapeDtypeStruct((NC, n, d), jnp.float32),
                  jax.ShapeDtypeStruct((NC, n, LANES), jnp.float32)),
        mesh=mesh,
        compiler_params=pltpu.CompilerParams(
            use_tc_tiling_on_sc=False, needs_layout_passes=False),
        scratch_types=[
            pltpu.VMEM((stage, chunk), jnp.int32),
            pltpu.VMEM((stage, chunk), jnp.int32),
            pltpu.VMEM((stage, chunk), jnp.float32),
            pltpu.VMEM((3, chunk, d), jnp.float32),
            pltpu.VMEM((chunk, LANES), jnp.float32),
            pltpu.VMEM_SHARED((n, d), jnp.float32),
            pltpu.VMEM_SHARED((n, LANES), jnp.float32),
            pltpu.SemaphoreType.DMA,
            pltpu.SemaphoreType.DMA,
            pltpu.SemaphoreType.DMA,
            pltpu.SemaphoreType.DMA,
            pltpu.SemaphoreType.DMA,
            pltpu.SemaphoreType.DMA,
            pltpu.SemaphoreType.DMA,
        ],
    )
    def sc_kernel(feat_hbm, src_hbm, dst_hbm, w_hbm, zs_hbm, zd_hbm,
                  out_s_hbm, out_d_hbm,
                  src_v, dst_v, w_v, rows_v, ones_v, acc_sh, deg_sh,
                  semg0, semg1, semg2, sems0, sems1, sems2, semdeg):
        cid = lax.axis_index("c")
        sid = lax.axis_index("s")
        wid = sid * NC + cid
        semg = (semg0, semg1, semg2)
        semsc = (sems0, sems1, sems2)

        ones16 = jnp.where(lax.iota(jnp.int32, LANES) == 0,
                           jnp.float32(1.0), jnp.float32(0.0))

        def init_ones(k, carry):
            ones_v[k, pl.ds(0, LANES)] = ones16
            return carry
        lax.fori_loop(0, chunk, init_ones, None)

        r0 = sid * rows_pt
        pltpu.sync_copy(zs_hbm.at[pl.ds(r0, rows_pt), :],
                        acc_sh.at[pl.ds(r0, rows_pt), :])
        pltpu.sync_copy(zd_hbm.at[pl.ds(r0, rows_pt), :],
                        deg_sh.at[pl.ds(r0, rows_pt), :])
        plsc.subcore_barrier()

        def issue_gather(ci, b):
            pltpu.async_copy(feat_hbm.at[src_v.at[ci]], rows_v.at[b], semg[b])

        def wait_gather(ci, b):
            pltpu.make_async_copy(feat_hbm.at[src_v.at[ci]],
                                  rows_v.at[b], semg[b]).wait()

        def issue_scatter(ci, b):
            pltpu.async_copy(ones_v, deg_sh.at[dst_v.at[ci]], semdeg, add=True)
            pltpu.async_copy(rows_v.at[b], acc_sh.at[dst_v.at[ci]],
                             semsc[b], add=True)

        def wait_scatter(ci, b):
            pltpu.make_async_copy(rows_v.at[b], acc_sh.at[dst_v.at[ci]],
                                  semsc[b]).wait()

        def compute(ci, b):
            def group_body(g, carry):
                k0 = g * LANES
                wg = w_v[ci, pl.ds(k0, LANES)]
                for j in range(LANES):
                    wb = lax.gather(
                        wg, jnp.full((LANES, 1), j, jnp.int32),
                        lax.GatherDimensionNumbers(
                            offset_dims=(), collapsed_slice_dims=(0,),
                            start_index_map=(0,)),
                        (1,), mode=lax.GatherScatterMode.PROMISE_IN_BOUNDS)
                    k = k0 + j
                    for jj in range(jblocks):
                        rows_v[b, k, pl.ds(jj * LANES, LANES)] = (
                            rows_v[b, k, pl.ds(jj * LANES, LANES)] * wb)
                return carry
            lax.fori_loop(0, chunk // LANES, group_body, None)

        def step(ci, b, first=False):
            wait_gather(ci, b)
            compute(ci, b)
            issue_scatter(ci, b)
            if not first:
                wait_scatter(ci - 1, (b + 2) % 3)

            @pl.when(ci + 2 < stage)
            def _():
                issue_gather(ci + 2, (b + 2) % 3)

        def stage_body(s, carry):
            sb = s * stage
            pltpu.sync_copy(src_hbm.at[wid, pl.ds(sb, stage), :], src_v)
            pltpu.sync_copy(dst_hbm.at[wid, pl.ds(sb, stage), :], dst_v)
            pltpu.sync_copy(w_hbm.at[wid, pl.ds(sb, stage), :], w_v)

            issue_gather(0, 0)
            issue_gather(1, 1)
            step(0, 0, first=True)
            step(1, 1)

            def triple_body(t, c2):
                ci0 = 3 * t + 2
                step(ci0, 2)
                step(ci0 + 1, 0)
                step(ci0 + 2, 1)
                return c2
            lax.fori_loop(0, (stage - 4) // 3, triple_body, None)

            step(stage - 2, (stage - 2) % 3)
            step(stage - 1, (stage - 1) % 3)
            wait_scatter(stage - 1, (stage - 1) % 3)

            def deg_drain(ci, c2):
                pltpu.make_async_copy(ones_v, deg_sh.at[dst_v.at[0]],
                                      semdeg).wait()
                return c2
            lax.fori_loop(0, stage, deg_drain, None)
            return carry
        lax.fori_loop(0, nstage, stage_body, None)

        plsc.subcore_barrier()
        pltpu.sync_copy(acc_sh.at[pl.ds(r0, rows_pt), :],
                        out_s_hbm.at[cid, pl.ds(r0, rows_pt), :])
        pltpu.sync_copy(deg_sh.at[pl.ds(r0, rows_pt), :],
                        out_d_hbm.at[cid, pl.ds(r0, rows_pt), :])

    return sc_kernel(feature, srcr, dstr, wr, zeros_s, zeros_d)


def _tc_combine(n, d, psum, pdeg, feature, w1t, w2t, b2d):
    blk = 1000

    def body(ps_ref, pd_ref, f_ref, w1_ref, w2_ref, b_ref, o_ref):
        s = ps_ref[0] + ps_ref[1]
        deg = pd_ref[0, :, 0:1] + pd_ref[1, :, 0:1]
        r = s / jnp.maximum(deg, 1.0)
        acc = jnp.dot(f_ref[...], w1_ref[...],
                      preferred_element_type=jnp.float32)
        acc = acc + jnp.dot(r, w2_ref[...],
                            preferred_element_type=jnp.float32)
        o_ref[...] = jnp.maximum(acc + b_ref[...], 0.0)

    return pl.pallas_call(
        body,
        grid=(n // blk,),
        in_specs=[
            pl.BlockSpec((NC, blk, d), lambda i: (0, i, 0)),
            pl.BlockSpec((NC, blk, LANES), lambda i: (0, i, 0)),
            pl.BlockSpec((blk, d), lambda i: (i, 0)),
            pl.BlockSpec((d, d), lambda i: (0, 0)),
            pl.BlockSpec((d, d), lambda i: (0, 0)),
            pl.BlockSpec((1, d), lambda i: (0, 0)),
        ],
        out_specs=pl.BlockSpec((blk, d), lambda i: (i, 0)),
        out_shape=jax.ShapeDtypeStruct((n, d), jnp.float32),
    )(psum, pdeg, feature, w1t, w2t, b2d)


@jax.jit
def kernel(feature, edge_index, edge_weight, W, b):
    n, d = feature.shape
    e = edge_index.shape[1]
    src = edge_index[0]
    dst = edge_index[1]

    c = feature[src[0]]
    w1t = W[:, :d].T
    w2t = (W[:, d:] * c[None, :]).T
    b2d = b.reshape(1, d)
    zeros_s = jnp.zeros((n, d), jnp.float32)
    zeros_d = jnp.zeros((n, LANES), jnp.float32)

    epw = e // NW
    chunk = 80
    srcr = src.reshape(NW, epw // chunk, chunk)
    dstr = dst.reshape(NW, epw // chunk, chunk)
    wr = edge_weight.reshape(NW, epw // chunk, chunk)

    psum, pdeg = _sc_partials(n, e, d, feature, srcr, dstr, wr,
                              zeros_s, zeros_d)
    return _tc_combine(n, d, psum, pdeg, feature, w1t, w2t, b2d)


# speedup vs baseline: 13.5764x; 1.0678x over previous
import functools

import jax
import jax.numpy as jnp
from jax import lax
from jax.experimental import pallas as pl
from jax.experimental.pallas import tpu as pltpu
from jax.experimental.pallas import tpu_sc as plsc

NC = 2
NS = 16
NW = NC * NS
LANES = 16
ROWLEN = 144


def _sc_partials(n, e, d, feature, srcr, dstr, wr):
    epw = e // NW
    chunk = 80
    nchunk = epw // chunk
    stage = 25
    nstage = nchunk // stage
    npair = stage // 2
    rows_pt = n // NS
    jblocks = d // LANES

    mesh = plsc.VectorSubcoreMesh(core_axis_name="c", subcore_axis_name="s")

    @functools.partial(
        pl.kernel,
        out_type=(jax.ShapeDtypeStruct((NC, n, d), jnp.float32),
                  jax.ShapeDtypeStruct((NC, n, LANES), jnp.float32)),
        mesh=mesh,
        compiler_params=pltpu.CompilerParams(
            use_tc_tiling_on_sc=False, needs_layout_passes=False),
        scratch_types=[
            pltpu.VMEM((stage, chunk), jnp.int32),
            pltpu.VMEM((stage, chunk), jnp.int32),
            pltpu.VMEM((stage, chunk), jnp.float32),
            pltpu.VMEM((3, chunk, d), jnp.float32),
            pltpu.VMEM((chunk, LANES), jnp.float32),
            pltpu.VMEM((chunk, LANES), jnp.float32),
            pltpu.VMEM_SHARED((n, d), jnp.float32),
            pltpu.VMEM_SHARED((n, LANES), jnp.float32),
            pltpu.SemaphoreType.DMA,
            pltpu.SemaphoreType.DMA,
            pltpu.SemaphoreType.DMA,
            pltpu.SemaphoreType.DMA,
            pltpu.SemaphoreType.DMA,
            pltpu.SemaphoreType.DMA,
            pltpu.SemaphoreType.DMA,
            pltpu.SemaphoreType.DMA,
        ],
    )
    def sc_kernel(feat_hbm, src_hbm, dst_hbm, w_hbm,
                  out_s_hbm, out_d_hbm,
                  src_v, dst_v, w_v, rows_v, ones_v, zero_v, acc_sh, deg_sh,
                  semg0, semg1, semg2, sems0, sems1, sems2, semdeg, semidx):
        cid = lax.axis_index("c")
        sid = lax.axis_index("s")
        wid = sid * NC + cid
        semg = (semg0, semg1, semg2)
        semsc = (sems0, sems1, sems2)

        ones16 = jnp.where(lax.iota(jnp.int32, LANES) == 0,
                           jnp.float32(1.0), jnp.float32(0.0))
        zero16 = jnp.zeros((LANES,), jnp.float32)

        def init_consts(k, carry):
            ones_v[k, pl.ds(0, LANES)] = ones16
            zero_v[k, pl.ds(0, LANES)] = zero16
            for j in range(jblocks):
                rows_v[0, k, pl.ds(j * LANES, LANES)] = zero16
            return carry
        lax.fori_loop(0, chunk, init_consts, None)

        r0 = sid * rows_pt
        nslab = rows_pt // chunk
        rem = rows_pt - nslab * chunk

        def zero_slab(i, carry):
            pltpu.sync_copy(rows_v.at[0],
                            acc_sh.at[pl.ds(r0 + i * chunk, chunk), :])
            pltpu.sync_copy(zero_v,
                            deg_sh.at[pl.ds(r0 + i * chunk, chunk), :])
            return carry
        lax.fori_loop(0, nslab, zero_slab, None)
        if rem:
            pltpu.sync_copy(rows_v.at[0, pl.ds(0, rem), :],
                            acc_sh.at[pl.ds(r0 + nslab * chunk, rem), :])
            pltpu.sync_copy(zero_v.at[pl.ds(0, rem), :],
                            deg_sh.at[pl.ds(r0 + nslab * chunk, rem), :])
        plsc.subcore_barrier()

        def issue_gather(ci, b):
            pltpu.async_copy(feat_hbm.at[src_v.at[ci]], rows_v.at[b], semg[b])

        def wait_gather(ci, b):
            pltpu.make_async_copy(feat_hbm.at[src_v.at[ci]],
                                  rows_v.at[b], semg[b]).wait()

        def issue_scatter(ci, b):
            pltpu.async_copy(ones_v, deg_sh.at[dst_v.at[ci]], semdeg, add=True)
            pltpu.async_copy(rows_v.at[b], acc_sh.at[dst_v.at[ci]],
                             semsc[b], add=True)

        def wait_scatter(ci, b):
            pltpu.make_async_copy(rows_v.at[b], acc_sh.at[dst_v.at[ci]],
                                  semsc[b]).wait()

        def compute(ci, b):
            def group_body(g, carry):
                k0 = g * LANES
                wg = w_v[ci, pl.ds(k0, LANES)]
                for j in range(LANES):
                    wb = lax.gather(
                        wg, jnp.full((LANES, 1), j, jnp.int32),
                        lax.GatherDimensionNumbers(
                            offset_dims=(), collapsed_slice_dims=(0,),
                            start_index_map=(0,)),
                        (1,), mode=lax.GatherScatterMode.PROMISE_IN_BOUNDS)
                    k = k0 + j
                    for jj in range(jblocks):
                        rows_v[b, k, pl.ds(jj * LANES, LANES)] = (
                            rows_v[b, k, pl.ds(jj * LANES, LANES)] * wb)
                return carry
            lax.fori_loop(0, chunk // LANES, group_body, None)

        def step(ci, b, first=False):
            wait_gather(ci, b)
            compute(ci, b)
            issue_scatter(ci, b)
            if not first:
                wait_scatter(ci - 1, (b + 2) % 3)

            @pl.when(ci + 2 < stage)
            def _():
                issue_gather(ci + 2, (b + 2) % 3)

        def stage_body(s, carry):
            sb = s * stage
            pltpu.async_copy(src_hbm.at[wid, pl.ds(sb, stage), :], src_v,
                             semidx)
            pltpu.async_copy(dst_hbm.at[wid, pl.ds(sb, stage), :], dst_v,
                             semidx)
            pltpu.async_copy(w_hbm.at[wid, pl.ds(sb, stage), :], w_v, semidx)
            pltpu.make_async_copy(src_hbm.at[wid, pl.ds(sb, stage), :],
                                  src_v, semidx).wait()
            pltpu.make_async_copy(dst_hbm.at[wid, pl.ds(sb, stage), :],
                                  dst_v, semidx).wait()
            pltpu.make_async_copy(w_hbm.at[wid, pl.ds(sb, stage), :],
                                  w_v, semidx).wait()

            issue_gather(0, 0)
            issue_gather(1, 1)
            step(0, 0, first=True)
            step(1, 1)

            def triple_body(t, c2):
                ci0 = 3 * t + 2
                step(ci0, 2)
                step(ci0 + 1, 0)
                step(ci0 + 2, 1)
                return c2
            lax.fori_loop(0, (stage - 4) // 3, triple_body, None)

            step(stage - 2, (stage - 2) % 3)
            step(stage - 1, (stage - 1) % 3)
            wait_scatter(stage - 1, (stage - 1) % 3)

            def deg_drain(ci, c2):
                pltpu.make_async_copy(ones_v, deg_sh.at[dst_v.at[0]],
                                      semdeg).wait()
                return c2
            lax.fori_loop(0, stage, deg_drain, None)
            return carry
        lax.fori_loop(0, nstage, stage_body, None)

        plsc.subcore_barrier()
        pltpu.sync_copy(acc_sh.at[pl.ds(r0, rows_pt), :],
                        out_s_hbm.at[cid, pl.ds(r0, rows_pt), :])
        pltpu.sync_copy(deg_sh.at[pl.ds(r0, rows_pt), :],
                        out_d_hbm.at[cid, pl.ds(r0, rows_pt), :])

    return sc_kernel(feature, srcr, dstr, wr)


def _tc_combine(n, d, psum, pdeg, feature, w1t, w2t, b2d):
    blk = 1000

    def body(ps_ref, pd_ref, f_ref, w1_ref, w2_ref, b_ref, o_ref):
        s = ps_ref[0] + ps_ref[1]
        deg = pd_ref[0, :, 0:1] + pd_ref[1, :, 0:1]
        r = s / jnp.maximum(deg, 1.0)
        acc = jnp.dot(f_ref[...], w1_ref[...],
                      preferred_element_type=jnp.float32)
        acc = acc + jnp.dot(r, w2_ref[...],
                            preferred_element_type=jnp.float32)
        o_ref[...] = jnp.maximum(acc + b_ref[...], 0.0)

    return pl.pallas_call(
        body,
        grid=(n // blk,),
        in_specs=[
            pl.BlockSpec((NC, blk, d), lambda i: (0, i, 0)),
            pl.BlockSpec((NC, blk, LANES), lambda i: (0, i, 0)),
            pl.BlockSpec((blk, d), lambda i: (i, 0)),
            pl.BlockSpec((d, d), lambda i: (0, 0)),
            pl.BlockSpec((d, d), lambda i: (0, 0)),
            pl.BlockSpec((1, d), lambda i: (0, 0)),
        ],
        out_specs=pl.BlockSpec((blk, d), lambda i: (i, 0)),
        out_shape=jax.ShapeDtypeStruct((n, d), jnp.float32),
    )(psum, pdeg, feature, w1t, w2t, b2d)


@jax.jit
def kernel(feature, edge_index, edge_weight, W, b):
    n, d = feature.shape
    e = edge_index.shape[1]
    src = edge_index[0]
    dst = edge_index[1]

    c = feature[src[0]]
    w1t = W[:, :d].T
    w2t = (W[:, d:] * c[None, :]).T
    b2d = b.reshape(1, d)

    epw = e // NW
    chunk = 80
    srcr = src.reshape(NW, epw // chunk, chunk)
    dstr = dst.reshape(NW, epw // chunk, chunk)
    wr = edge_weight.reshape(NW, epw // chunk, chunk)

    psum, pdeg = _sc_partials(n, e, d, feature, srcr, dstr, wr)
    return _tc_combine(n, d, psum, pdeg, feature, w1t, w2t, b2d)
